# baseline (speedup 1.0000x reference)
import jax
import jax.numpy as jnp
from jax.experimental import pallas as pl
from jax.experimental.pallas import tpu as pltpu

N = 20000
R, C = 160, 128
P = R * C
POOL_ROWS = 8
IMG_H, IMG_W = 512.0, 512.0
BOX_FILTER_THRESHOLD = 0.05
NMS_THRESHOLD = 0.5
POST_NMS_TOP_K = 100
NEG_INF = -1e9


def _nms_kernel(cls_ref, del_ref, anc_ref,
                box_out, sc_out, lb_out,
                x1_ref, y1_ref, x2_ref, y2_ref, area_ref, lab_ref,
                sw_ref, kept_ref,
                psw_ref, pidx_ref, px1_ref, py1_ref, px2_ref, py2_ref,
                parea_ref, plab_ref,
                tau_ref, lastt_ref):
    num_classes = cls_ref.shape[0]

    row_iota = jax.lax.broadcasted_iota(jnp.int32, (R, C), 0)
    col_iota = jax.lax.broadcasted_iota(jnp.int32, (R, C), 1)
    lin = row_iota * C + col_iota
    lane_iota = jax.lax.broadcasted_iota(jnp.int32, (1, C), 1)
    slin = (jax.lax.broadcasted_iota(jnp.int32, (8, 128), 0) * 128
            + jax.lax.broadcasted_iota(jnp.int32, (8, 128), 1))

    def cls_body(c, carry):
        best, lab = carry
        v = cls_ref[c]
        better = v > best
        return jnp.where(better, v, best), jnp.where(better, c, lab)

    best, labels = jax.lax.fori_loop(
        1, num_classes, cls_body, (cls_ref[0], jnp.zeros((R, C), jnp.int32)))
    lab_ref[...] = labels

    ax, ay, aw, ah = anc_ref[0], anc_ref[1], anc_ref[2], anc_ref[3]
    dx, dy, dw, dh = del_ref[0], del_ref[1], del_ref[2], del_ref[3]
    cx = ax + dx * aw
    cy = ay + dy * ah
    w = aw * jnp.exp(dw)
    h = ah * jnp.exp(dh)
    x1 = jnp.clip(cx - 0.5 * w, 0.0, IMG_W)
    y1 = jnp.clip(cy - 0.5 * h, 0.0, IMG_H)
    x2 = jnp.clip(cx + 0.5 * w, 0.0, IMG_W)
    y2 = jnp.clip(cy + 0.5 * h, 0.0, IMG_H)
    x1_ref[...] = x1
    y1_ref[...] = y1
    x2_ref[...] = x2
    y2_ref[...] = y2
    area_ref[...] = jnp.maximum(x2 - x1, 0.0) * jnp.maximum(y2 - y1, 0.0)

    sw_ref[...] = jnp.where(best >= BOX_FILTER_THRESHOLD, best, NEG_INF)

    sc_out[...] = jnp.zeros((8, 128), jnp.float32)
    lb_out[...] = jnp.full((8, 128), -1, jnp.int32)
    for i in range(4):
        box_out[i] = jnp.zeros((8, 128), jnp.float32)
    kept_ref[...] = jnp.full((8, 128), -1, jnp.int32)
    lastt_ref[0] = 0

    def build_pool():
        work = sw_ref[...]
        for r in range(POOL_ROWS):
            m = jnp.max(work, axis=0, keepdims=True)
            sel_row = jnp.min(jnp.where(work == m, row_iota, R),
                              axis=0, keepdims=True)
            mask = row_iota == sel_row
            psw_ref[r:r + 1, :] = m
            pidx_ref[r:r + 1, :] = sel_row * C + lane_iota
            px1_ref[r:r + 1, :] = jnp.sum(jnp.where(mask, x1_ref[...], 0.0),
                                          axis=0, keepdims=True)
            py1_ref[r:r + 1, :] = jnp.sum(jnp.where(mask, y1_ref[...], 0.0),
                                          axis=0, keepdims=True)
            px2_ref[r:r + 1, :] = jnp.sum(jnp.where(mask, x2_ref[...], 0.0),
                                          axis=0, keepdims=True)
            py2_ref[r:r + 1, :] = jnp.sum(jnp.where(mask, y2_ref[...], 0.0),
                                          axis=0, keepdims=True)
            parea_ref[r:r + 1, :] = jnp.sum(
                jnp.where(mask, area_ref[...], 0.0), axis=0, keepdims=True)
            plab_ref[r:r + 1, :] = jnp.sum(jnp.where(mask, lab_ref[...], 0),
                                           axis=0, keepdims=True)
            work = jnp.where(mask, -jnp.inf, work)
        tau_ref[0] = jnp.max(work)

    build_pool()

    def emit(t, valid, s, bx1, by1, bx2, by2, blab, pick):
        hot_t = slin == t
        sc_out[...] = jnp.where(hot_t, jnp.where(valid, s, 0.0), sc_out[...])
        lb_out[...] = jnp.where(hot_t, jnp.where(valid, blab, -1), lb_out[...])
        bvals = (bx1, by1, bx2, by2)
        for i in range(4):
            box_out[i] = jnp.where(hot_t, jnp.where(valid, bvals[i], 0.0),
                                   box_out[i])
        kept_ref[...] = jnp.where(hot_t & valid, pick, kept_ref[...])

    def pool_step(t):
        psw = psw_ref[...]
        pidx = pidx_ref[...]
        s = jnp.max(psw)
        pick = jnp.min(jnp.where(psw == s, pidx, jnp.int32(P)))
        hot = pidx == pick
        bx1 = jnp.sum(jnp.where(hot, px1_ref[...], 0.0))
        by1 = jnp.sum(jnp.where(hot, py1_ref[...], 0.0))
        bx2 = jnp.sum(jnp.where(hot, px2_ref[...], 0.0))
        by2 = jnp.sum(jnp.where(hot, py2_ref[...], 0.0))
        blab = jnp.sum(jnp.where(hot, plab_ref[...], 0))
        area_a = jnp.maximum(bx2 - bx1, 0.0) * jnp.maximum(by2 - by1, 0.0)
        valid = s > (NEG_INF / 2.0)

        inter = (jnp.maximum(jnp.minimum(bx2, px2_ref[...])
                             - jnp.maximum(bx1, px1_ref[...]), 0.0)
                 * jnp.maximum(jnp.minimum(by2, py2_ref[...])
                               - jnp.maximum(by1, py1_ref[...]), 0.0))
        iou = inter / (area_a + parea_ref[...] - inter + 1e-9)
        psw_ref[...] = jnp.where(((iou > NMS_THRESHOLD) & valid) | hot,
                                 NEG_INF, psw)
        emit(t, valid, s, bx1, by1, bx2, by2, blab, pick)

    def apply_kept(tp):
        hot = slin == tp
        kx1 = jnp.sum(jnp.where(hot, box_out[0], 0.0))
        ky1 = jnp.sum(jnp.where(hot, box_out[1], 0.0))
        kx2 = jnp.sum(jnp.where(hot, box_out[2], 0.0))
        ky2 = jnp.sum(jnp.where(hot, box_out[3], 0.0))
        klin = jnp.sum(jnp.where(hot, kept_ref[...], 0))
        karea = jnp.maximum(kx2 - kx1, 0.0) * jnp.maximum(ky2 - ky1, 0.0)
        inter = (jnp.maximum(jnp.minimum(kx2, x2_ref[...])
                             - jnp.maximum(kx1, x1_ref[...]), 0.0)
                 * jnp.maximum(jnp.minimum(ky2, y2_ref[...])
                               - jnp.maximum(ky1, y1_ref[...]), 0.0))
        iou = inter / (karea + area_ref[...] - inter + 1e-9)
        sw_ref[...] = jnp.where((iou > NMS_THRESHOLD) | (lin == klin),
                                NEG_INF, sw_ref[...])

    def full_step(t):
        sw = sw_ref[...]
        s = jnp.max(sw)
        idx = jnp.min(jnp.where(sw == s, lin, jnp.int32(P)))
        row = idx // C
        lane_hot = lane_iota == idx - row * C

        def ext(ref, zero):
            return jnp.sum(jnp.where(lane_hot, ref[pl.ds(row, 1), :], zero))

        bx1 = ext(x1_ref, 0.0)
        by1 = ext(y1_ref, 0.0)
        bx2 = ext(x2_ref, 0.0)
        by2 = ext(y2_ref, 0.0)
        blab = ext(lab_ref, 0)
        area_a = jnp.maximum(bx2 - bx1, 0.0) * jnp.maximum(by2 - by1, 0.0)
        valid = s > (NEG_INF / 2.0)

        inter = (jnp.maximum(jnp.minimum(bx2, x2_ref[...])
                             - jnp.maximum(bx1, x1_ref[...]), 0.0)
                 * jnp.maximum(jnp.minimum(by2, y2_ref[...])
                               - jnp.maximum(by1, y1_ref[...]), 0.0))
        iou = inter / (area_a + area_ref[...] - inter + 1e-9)
        sw_ref[...] = jnp.where(((iou > NMS_THRESHOLD) & valid) | (lin == idx),
                                NEG_INF, sw)
        emit(t, valid, s, bx1, by1, bx2, by2, blab, idx)

    def refill(t):
        jax.lax.fori_loop(lastt_ref[0], t,
                          lambda tp, c: (apply_kept(tp), c)[1], 0)
        full_step(t)
        build_pool()
        lastt_ref[0] = t + 1

    def body(t, carry):
        tau = tau_ref[0]
        need = (jnp.max(psw_ref[...]) <= tau) & (tau > (NEG_INF / 2.0))

        @pl.when(need)
        def _():
            refill(t)

        @pl.when(jnp.logical_not(need))
        def _():
            pool_step(t)

        return carry

    jax.lax.fori_loop(0, POST_NMS_TOP_K, body, 0)


def kernel(cls_scores, box_deltas, anchors):
    n, num_classes = cls_scores.shape
    pad = P - n
    cls_t = jnp.pad(cls_scores, ((0, pad), (0, 0)),
                    constant_values=-1.0).T.reshape(num_classes, R, C)
    del_t = jnp.pad(box_deltas, ((0, pad), (0, 0))).T.reshape(4, R, C)
    anc_t = jnp.pad(anchors, ((0, pad), (0, 0))).T.reshape(4, R, C)

    f32, i32 = jnp.float32, jnp.int32
    bx, sc, lb = pl.pallas_call(
        _nms_kernel,
        out_shape=(
            jax.ShapeDtypeStruct((4, 8, 128), f32),
            jax.ShapeDtypeStruct((8, 128), f32),
            jax.ShapeDtypeStruct((8, 128), i32),
        ),
        scratch_shapes=[
            pltpu.VMEM((R, C), f32),
            pltpu.VMEM((R, C), f32),
            pltpu.VMEM((R, C), f32),
            pltpu.VMEM((R, C), f32),
            pltpu.VMEM((R, C), f32),
            pltpu.VMEM((R, C), i32),
            pltpu.VMEM((R, C), f32),
            pltpu.VMEM((8, 128), i32),
            pltpu.VMEM((POOL_ROWS, 128), f32),
            pltpu.VMEM((POOL_ROWS, 128), i32),
            pltpu.VMEM((POOL_ROWS, 128), f32),
            pltpu.VMEM((POOL_ROWS, 128), f32),
            pltpu.VMEM((POOL_ROWS, 128), f32),
            pltpu.VMEM((POOL_ROWS, 128), f32),
            pltpu.VMEM((POOL_ROWS, 128), f32),
            pltpu.VMEM((POOL_ROWS, 128), i32),
            pltpu.SMEM((1,), f32),
            pltpu.SMEM((1,), i32),
        ],
    )(cls_t, del_t, anc_t)

    boxes = bx.reshape(4, 8 * 128)[:, :POST_NMS_TOP_K].T
    scores = sc.reshape(8 * 128)[:POST_NMS_TOP_K]
    labels = lb.reshape(8 * 128)[:POST_NMS_TOP_K]
    return boxes, scores, labels
